# Initial kernel scaffold; baseline (speedup 1.0000x reference)
import jax
import jax.numpy as jnp
from jax.experimental import pallas as pl


def kernel(X, edge_index, W1, b1, W2, b2):
    raise NotImplementedError("write your pallas kernel here")


# speedup vs baseline: 11.0609x; 11.0609x over previous
import functools

import jax
import jax.numpy as jnp
from jax import lax
from jax.experimental import pallas as pl
from jax.experimental.pallas import tpu as pltpu
from jax.experimental.pallas import tpu_sc as plsc

_NC = 2
_NS = 16
_NW = _NC * _NS
_B = 128


def _sc_mesh():
    return plsc.VectorSubcoreMesh(core_axis_name="c", subcore_axis_name="s")


def _sc_degree(dst_idx, zeros_cols, n_pad):
    nb = dst_idx.shape[1]
    chunk = n_pad // _NS

    @functools.partial(
        pl.kernel,
        out_type=jax.ShapeDtypeStruct((_NC, n_pad, 16), jnp.float32),
        mesh=_sc_mesh(),
        scratch_types=[
            pltpu.VMEM((nb, _B), jnp.int32),
            pltpu.VMEM((_B, 16), jnp.float32),
            pltpu.VMEM_SHARED((n_pad, 16), jnp.float32),
        ],
    )
    def k(dst_hbm, z_hbm, deg_out, dst_v, ones_v, deg_sh):
        c = lax.axis_index("c")
        s = lax.axis_index("s")
        wid = s * _NC + c
        pltpu.sync_copy(dst_hbm.at[wid], dst_v)

        def fill(i, carry):
            ones_v[i] = jnp.ones((16,), jnp.float32)
            return carry

        lax.fori_loop(0, _B, fill, 0)
        pltpu.sync_copy(z_hbm.at[pl.ds(s * chunk, chunk)],
                        deg_sh.at[pl.ds(s * chunk, chunk)])
        plsc.subcore_barrier()

        def body(j, carry):
            pltpu.sync_copy(ones_v, deg_sh.at[dst_v.at[j]], add=True)
            return carry

        lax.fori_loop(0, nb, body, 0)
        plsc.subcore_barrier()
        pltpu.sync_copy(deg_sh.at[pl.ds(s * chunk, chunk)],
                        deg_out.at[c, pl.ds(s * chunk, chunk)])

    return k(dst_idx, zeros_cols)


def _sc_aggregate(y, src_idx, dst_idx, zeros_rows, n_pad):
    nb = src_idx.shape[1]
    d = y.shape[1]
    chunk = n_pad // _NS

    @functools.partial(
        pl.kernel,
        out_type=jax.ShapeDtypeStruct((_NC, n_pad, d), jnp.float32),
        mesh=_sc_mesh(),
        scratch_types=[
            pltpu.VMEM((nb, _B), jnp.int32),
            pltpu.VMEM((nb, _B), jnp.int32),
            pltpu.VMEM((_B, d), jnp.float32),
            pltpu.VMEM_SHARED((n_pad, d), jnp.float32),
        ],
    )
    def k(y_hbm, src_hbm, dst_hbm, z_hbm, agg_out, src_v, dst_v, buf, agg_sh):
        c = lax.axis_index("c")
        s = lax.axis_index("s")
        wid = s * _NC + c
        pltpu.sync_copy(src_hbm.at[wid], src_v)
        pltpu.sync_copy(dst_hbm.at[wid], dst_v)
        pltpu.sync_copy(z_hbm.at[pl.ds(s * chunk, chunk)],
                        agg_sh.at[pl.ds(s * chunk, chunk)])
        plsc.subcore_barrier()

        def body(j, carry):
            pltpu.sync_copy(y_hbm.at[src_v.at[j]], buf)
            pltpu.sync_copy(buf, agg_sh.at[dst_v.at[j]], add=True)
            return carry

        lax.fori_loop(0, nb, body, 0)
        plsc.subcore_barrier()
        pltpu.sync_copy(agg_sh.at[pl.ds(s * chunk, chunk)],
                        agg_out.at[c, pl.ds(s * chunk, chunk)])

    return k(y, src_idx, dst_idx, zeros_rows)


def _tc_stage1(X, W1, degp, bn):
    n, din = X.shape
    dh = W1.shape[1]
    grid = n // bn

    def body(x_ref, w_ref, deg_ref, xw_ref, y_ref, dis_ref, dinv_ref):
        deg = deg_ref[0] + deg_ref[1] + 1.0
        dis = lax.rsqrt(deg)
        dinv = 1.0 / deg
        xw = jnp.dot(x_ref[...], w_ref[...], preferred_element_type=jnp.float32)
        xw_ref[...] = xw
        y_ref[...] = xw * dis
        dis_ref[...] = dis
        dinv_ref[...] = dinv

    return pl.pallas_call(
        body,
        grid=(grid,),
        in_specs=[
            pl.BlockSpec((bn, din), lambda i: (i, 0)),
            pl.BlockSpec((din, dh), lambda i: (0, 0)),
            pl.BlockSpec((_NC, bn, 1), lambda i: (0, i, 0)),
        ],
        out_specs=[
            pl.BlockSpec((bn, dh), lambda i: (i, 0)),
            pl.BlockSpec((bn, dh), lambda i: (i, 0)),
            pl.BlockSpec((bn, 1), lambda i: (i, 0)),
            pl.BlockSpec((bn, 1), lambda i: (i, 0)),
        ],
        out_shape=[
            jax.ShapeDtypeStruct((n, dh), jnp.float32),
            jax.ShapeDtypeStruct((n, dh), jnp.float32),
            jax.ShapeDtypeStruct((n, 1), jnp.float32),
            jax.ShapeDtypeStruct((n, 1), jnp.float32),
        ],
    )(X, W1, degp)


def _tc_stage2(agg, xw1, dis, dinv, b1, W2, bn):
    n, dh = xw1.shape
    do = W2.shape[1]
    grid = n // bn

    def body(a_ref, xw_ref, dis_ref, dinv_ref, b_ref, w_ref, xw2_ref, y2_ref):
        h = (dis_ref[...] * (a_ref[0] + a_ref[1])
             + xw_ref[...] * dinv_ref[...] + b_ref[...])
        h = jnp.maximum(h, 0.0)
        xw2 = jnp.dot(h, w_ref[...], preferred_element_type=jnp.float32)
        xw2_ref[...] = xw2
        y2_ref[...] = xw2 * dis_ref[...]

    return pl.pallas_call(
        body,
        grid=(grid,),
        in_specs=[
            pl.BlockSpec((_NC, bn, dh), lambda i: (0, i, 0)),
            pl.BlockSpec((bn, dh), lambda i: (i, 0)),
            pl.BlockSpec((bn, 1), lambda i: (i, 0)),
            pl.BlockSpec((bn, 1), lambda i: (i, 0)),
            pl.BlockSpec((1, dh), lambda i: (0, 0)),
            pl.BlockSpec((dh, do), lambda i: (0, 0)),
        ],
        out_specs=[
            pl.BlockSpec((bn, do), lambda i: (i, 0)),
            pl.BlockSpec((bn, do), lambda i: (i, 0)),
        ],
        out_shape=[
            jax.ShapeDtypeStruct((n, do), jnp.float32),
            jax.ShapeDtypeStruct((n, do), jnp.float32),
        ],
    )(agg, xw1, dis, dinv, b1, W2)


def _tc_stage3(agg, xw2, dis, dinv, b2, bn):
    n, do = xw2.shape
    grid = n // bn

    def body(a_ref, xw_ref, dis_ref, dinv_ref, b_ref, o_ref):
        o_ref[...] = (dis_ref[...] * (a_ref[0] + a_ref[1])
                      + xw_ref[...] * dinv_ref[...] + b_ref[...])

    return pl.pallas_call(
        body,
        grid=(grid,),
        in_specs=[
            pl.BlockSpec((_NC, bn, do), lambda i: (0, i, 0)),
            pl.BlockSpec((bn, do), lambda i: (i, 0)),
            pl.BlockSpec((bn, 1), lambda i: (i, 0)),
            pl.BlockSpec((bn, 1), lambda i: (i, 0)),
            pl.BlockSpec((1, do), lambda i: (0, 0)),
        ],
        out_specs=pl.BlockSpec((bn, do), lambda i: (i, 0)),
        out_shape=jax.ShapeDtypeStruct((n, do), jnp.float32),
    )(agg, xw2, dis, dinv, b2)


def kernel(X, edge_index, W1, b1, W2, b2):
    n, din = X.shape
    e = edge_index.shape[1]
    dh = W1.shape[1]
    do = W2.shape[1]

    nb = -(-e // (_NW * _B))
    e_pad = _NW * nb * _B
    n_pad = -(-(n + 1) // 128) * 128

    src = edge_index[0].astype(jnp.int32)
    dst = edge_index[1].astype(jnp.int32)
    pad = e_pad - e
    src_p = jnp.concatenate([src, jnp.zeros((pad,), jnp.int32)])
    dst_p = jnp.concatenate([dst, jnp.full((pad,), n, jnp.int32)])
    src_p = src_p.reshape(_NW, nb, _B)
    dst_p = dst_p.reshape(_NW, nb, _B)

    zeros_cols = jnp.zeros((n_pad, 16), jnp.float32)
    zeros_rows = jnp.zeros((n_pad, dh), jnp.float32)

    bn = 1000 if n % 1000 == 0 else n

    degp = _sc_degree(dst_p, zeros_cols, n_pad)
    degs = degp[:, :n, 0:1]

    xw1, y1, dis, dinv = _tc_stage1(X, W1, degs, bn)
    agg1 = _sc_aggregate(y1, src_p, dst_p, zeros_rows, n_pad)[:, :n, :]
    xw2, y2 = _tc_stage2(agg1, xw1, dis, dinv, b1.reshape(1, dh), W2, bn)
    agg2 = _sc_aggregate(y2, src_p, dst_p, zeros_rows, n_pad)[:, :n, :]
    return _tc_stage3(agg2, xw2, dis, dinv, b2.reshape(1, do), bn)


# speedup vs baseline: 12.5640x; 1.1359x over previous
import functools

import jax
import jax.numpy as jnp
from jax import lax
from jax.experimental import pallas as pl
from jax.experimental.pallas import tpu as pltpu
from jax.experimental.pallas import tpu_sc as plsc

_NC = 2
_NS = 16
_NW = _NC * _NS
_B = 128


def _sc_mesh():
    return plsc.VectorSubcoreMesh(core_axis_name="c", subcore_axis_name="s")


def _sc_degree(dst_idx, zeros_cols, n_pad):
    nb = dst_idx.shape[1]
    chunk = n_pad // _NS

    @functools.partial(
        pl.kernel,
        out_type=jax.ShapeDtypeStruct((_NC, n_pad, 16), jnp.float32),
        mesh=_sc_mesh(),
        scratch_types=[
            pltpu.VMEM((nb, _B), jnp.int32),
            pltpu.VMEM((_B, 16), jnp.float32),
            pltpu.VMEM_SHARED((n_pad, 16), jnp.float32),
        ],
    )
    def k(dst_hbm, z_hbm, deg_out, dst_v, ones_v, deg_sh):
        c = lax.axis_index("c")
        s = lax.axis_index("s")
        wid = s * _NC + c
        pltpu.sync_copy(dst_hbm.at[wid], dst_v)

        def fill(i, carry):
            ones_v[i] = jnp.ones((16,), jnp.float32)
            return carry

        lax.fori_loop(0, _B, fill, 0)
        pltpu.sync_copy(z_hbm.at[pl.ds(s * chunk, chunk)],
                        deg_sh.at[pl.ds(s * chunk, chunk)])
        plsc.subcore_barrier()

        def body(j, carry):
            pltpu.sync_copy(ones_v, deg_sh.at[dst_v.at[j]], add=True)
            return carry

        lax.fori_loop(0, nb, body, 0)
        plsc.subcore_barrier()
        pltpu.sync_copy(deg_sh.at[pl.ds(s * chunk, chunk)],
                        deg_out.at[c, pl.ds(s * chunk, chunk)])

    return k(dst_idx, zeros_cols)


def _make_sc_aggregate(nb_a, nb_b, d, n_pad):
    chunk = n_pad // _NS

    @functools.partial(
        pl.kernel,
        out_type=jax.ShapeDtypeStruct((_NC, n_pad, d), jnp.float32),
        mesh=_sc_mesh(),
        scratch_types=[
            pltpu.VMEM((nb_a, _B), jnp.int32),
            pltpu.VMEM((nb_a, _B), jnp.int32),
            pltpu.VMEM((_B, d), jnp.float32),
            pltpu.VMEM_SHARED((n_pad, d), jnp.float32),
        ],
    )
    def k(y_hbm, src_hbm, dst_hbm, z_hbm, agg_out,
          src_v, dst_v, buf0, agg_sh):
        c = lax.axis_index("c")
        s = lax.axis_index("s")
        wid = s * _NC + c
        pltpu.sync_copy(src_hbm.at[wid], src_v)
        pltpu.sync_copy(dst_hbm.at[wid], dst_v)
        pltpu.sync_copy(z_hbm.at[pl.ds(s * chunk, chunk)],
                        agg_sh.at[pl.ds(s * chunk, chunk)])
        plsc.subcore_barrier()

        nbc = jnp.where(c == 0, nb_a, nb_b)

        def body(j, carry):
            pltpu.sync_copy(y_hbm.at[src_v.at[j]], buf0)
            pltpu.sync_copy(buf0, agg_sh.at[dst_v.at[j]], add=True)
            return carry

        lax.fori_loop(0, nbc, body, 0)
        plsc.subcore_barrier()
        pltpu.sync_copy(agg_sh.at[pl.ds(s * chunk, chunk)],
                        agg_out.at[c, pl.ds(s * chunk, chunk)])

    return k


def _tc_stage1(X, W1, degp, bn, n_pad):
    n, din = X.shape
    dh = W1.shape[1]
    grid = n_pad // bn

    def body(x_ref, w_ref, deg_ref, xw_ref, y_ref, dis_ref, dinv_ref):
        deg = deg_ref[0] + deg_ref[1] + 1.0
        dis = lax.rsqrt(deg)
        dinv = 1.0 / deg
        xw = jnp.dot(x_ref[...], w_ref[...], preferred_element_type=jnp.float32)
        xw_ref[...] = xw
        y_ref[...] = xw * dis
        dis_ref[...] = dis
        dinv_ref[...] = dinv

    return pl.pallas_call(
        body,
        grid=(grid,),
        in_specs=[
            pl.BlockSpec((bn, din), lambda i: (i, 0)),
            pl.BlockSpec((din, dh), lambda i: (0, 0)),
            pl.BlockSpec((_NC, bn, 1), lambda i: (0, i, 0)),
        ],
        out_specs=[
            pl.BlockSpec((bn, dh), lambda i: (i, 0)),
            pl.BlockSpec((bn, dh), lambda i: (i, 0)),
            pl.BlockSpec((bn, 1), lambda i: (i, 0)),
            pl.BlockSpec((bn, 1), lambda i: (i, 0)),
        ],
        out_shape=[
            jax.ShapeDtypeStruct((n, dh), jnp.float32),
            jax.ShapeDtypeStruct((n_pad, dh), jnp.float32),
            jax.ShapeDtypeStruct((n, 1), jnp.float32),
            jax.ShapeDtypeStruct((n, 1), jnp.float32),
        ],
    )(X, W1, degp)


def _tc_stage2(agg, xw1, dis, dinv, b1, W2, bn, n_pad):
    n, dh = xw1.shape
    do = W2.shape[1]
    grid = n_pad // bn

    def body(a_ref, xw_ref, dis_ref, dinv_ref, b_ref, w_ref, xw2_ref, y2_ref):
        h = (dis_ref[...] * (a_ref[0] + a_ref[1])
             + xw_ref[...] * dinv_ref[...] + b_ref[...])
        h = jnp.maximum(h, 0.0)
        xw2 = jnp.dot(h, w_ref[...], preferred_element_type=jnp.float32)
        xw2_ref[...] = xw2
        y2_ref[...] = xw2 * dis_ref[...]

    return pl.pallas_call(
        body,
        grid=(grid,),
        in_specs=[
            pl.BlockSpec((_NC, bn, dh), lambda i: (0, i, 0)),
            pl.BlockSpec((bn, dh), lambda i: (i, 0)),
            pl.BlockSpec((bn, 1), lambda i: (i, 0)),
            pl.BlockSpec((bn, 1), lambda i: (i, 0)),
            pl.BlockSpec((1, dh), lambda i: (0, 0)),
            pl.BlockSpec((dh, do), lambda i: (0, 0)),
        ],
        out_specs=[
            pl.BlockSpec((bn, do), lambda i: (i, 0)),
            pl.BlockSpec((bn, do), lambda i: (i, 0)),
        ],
        out_shape=[
            jax.ShapeDtypeStruct((n, do), jnp.float32),
            jax.ShapeDtypeStruct((n_pad, do), jnp.float32),
        ],
    )(agg, xw1, dis, dinv, b1, W2)


def _tc_stage3(agg, xw2, dis, dinv, b2, bn, n_pad):
    n, do = xw2.shape
    grid = n_pad // bn

    def body(a_ref, xw_ref, dis_ref, dinv_ref, b_ref, o_ref):
        o_ref[...] = (dis_ref[...] * (a_ref[0] + a_ref[1])
                      + xw_ref[...] * dinv_ref[...] + b_ref[...])

    return pl.pallas_call(
        body,
        grid=(grid,),
        in_specs=[
            pl.BlockSpec((_NC, bn, do), lambda i: (0, i, 0)),
            pl.BlockSpec((bn, do), lambda i: (i, 0)),
            pl.BlockSpec((bn, 1), lambda i: (i, 0)),
            pl.BlockSpec((bn, 1), lambda i: (i, 0)),
            pl.BlockSpec((1, do), lambda i: (0, 0)),
        ],
        out_specs=pl.BlockSpec((bn, do), lambda i: (i, 0)),
        out_shape=jax.ShapeDtypeStruct((n, do), jnp.float32),
    )(agg, xw2, dis, dinv, b2)


def kernel(X, edge_index, W1, b1, W2, b2):
    n, din = X.shape
    e = edge_index.shape[1]
    dh = W1.shape[1]
    do = W2.shape[1]

    nbw = -(-e // (_NW * _B))
    nbw += nbw % 2
    e_padw = _NW * nbw * _B
    per_tile = -(-e // (_NS * _B))
    nb_a = -(-2 * per_tile // 3)
    nb_a += nb_a % 2
    nb_b = max(per_tile - nb_a, 2)
    nb_b += nb_b % 2
    e_a = _NS * nb_a * _B
    e_ab = e_a + _NS * nb_b * _B
    n_pad = -(-(n + 1) // 128) * 128

    src = edge_index[0].astype(jnp.int32)
    dst = edge_index[1].astype(jnp.int32)
    dst_w = jnp.concatenate([dst, jnp.full((e_padw - e,), n, jnp.int32)])
    dst_w = dst_w.reshape(_NW, nbw, _B)
    src_p = jnp.concatenate([src, jnp.zeros((e_ab - e,), jnp.int32)])
    dst_p = jnp.concatenate([dst, jnp.full((e_ab - e,), n, jnp.int32)])

    def widmajor(x, fill):
        xa = x[:e_a].reshape(_NS, 1, nb_a, _B)
        xb = x[e_a:].reshape(_NS, nb_b, _B)
        xb = jnp.concatenate(
            [xb, jnp.full((_NS, nb_a - nb_b, _B), fill, jnp.int32)], axis=1)
        return jnp.concatenate([xa, xb[:, None]], axis=1).reshape(
            _NW, nb_a, _B)

    src_a = widmajor(src_p, 0)
    dst_a = widmajor(dst_p, n)

    zeros_cols = jnp.zeros((n_pad, 16), jnp.float32)
    zeros_rows = jnp.zeros((n_pad, dh), jnp.float32)

    bn = n_pad // _NS

    degp = _sc_degree(dst_w, zeros_cols, n_pad)
    degs = degp[:, :n, 0:1]

    agg_k = _make_sc_aggregate(nb_a, nb_b, dh, n_pad)

    xw1, y1, dis, dinv = _tc_stage1(X, W1, degs, bn, n_pad)
    agg1 = agg_k(y1, src_a, dst_a, zeros_rows)
    xw2, y2 = _tc_stage2(agg1, xw1, dis, dinv, b1.reshape(1, dh), W2, bn, n_pad)
    agg2 = agg_k(y2, src_a, dst_a, zeros_rows)
    return _tc_stage3(agg2, xw2, dis, dinv, b2.reshape(1, do), bn, n_pad)


# speedup vs baseline: 13.2835x; 1.0573x over previous
import functools

import jax
import jax.numpy as jnp
from jax import lax
from jax.experimental import pallas as pl
from jax.experimental.pallas import tpu as pltpu
from jax.experimental.pallas import tpu_sc as plsc

_NC = 2
_NS = 16
_NW = _NC * _NS
_B = 128


def _sc_mesh():
    return plsc.VectorSubcoreMesh(core_axis_name="c", subcore_axis_name="s")


def _sc_degree(dst_idx, zeros_cols, n_pad):
    nb = dst_idx.shape[1]
    chunk = n_pad // _NS

    @functools.partial(
        pl.kernel,
        out_type=jax.ShapeDtypeStruct((_NC, n_pad, 16), jnp.float32),
        mesh=_sc_mesh(),
        scratch_types=[
            pltpu.VMEM((nb, _B), jnp.int32),
            pltpu.VMEM((_B, 16), jnp.float32),
            pltpu.VMEM_SHARED((n_pad, 16), jnp.float32),
        ],
    )
    def k(dst_hbm, z_hbm, deg_out, dst_v, ones_v, deg_sh):
        c = lax.axis_index("c")
        s = lax.axis_index("s")
        wid = s * _NC + c
        pltpu.sync_copy(dst_hbm.at[wid], dst_v)

        def fill(i, carry):
            ones_v[i] = jnp.ones((16,), jnp.float32)
            return carry

        lax.fori_loop(0, _B, fill, 0)
        pltpu.sync_copy(z_hbm.at[pl.ds(s * chunk, chunk)],
                        deg_sh.at[pl.ds(s * chunk, chunk)])
        plsc.subcore_barrier()

        def body(j, carry):
            pltpu.sync_copy(ones_v, deg_sh.at[dst_v.at[j]], add=True)
            return carry

        lax.fori_loop(0, nb, body, 0)
        plsc.subcore_barrier()
        pltpu.sync_copy(deg_sh.at[pl.ds(s * chunk, chunk)],
                        deg_out.at[c, pl.ds(s * chunk, chunk)])

    return k(dst_idx, zeros_cols)


def _make_sc_aggregate(nb_a, nb_b, d, n_pad):
    chunk = n_pad // _NS

    @functools.partial(
        pl.kernel,
        out_type=jax.ShapeDtypeStruct((_NC, n_pad, d), jnp.float32),
        mesh=_sc_mesh(),
        scratch_types=[
            pltpu.VMEM((nb_a, _B), jnp.int32),
            pltpu.VMEM((nb_a, _B), jnp.int32),
            pltpu.VMEM((_B, d), jnp.float32),
            pltpu.VMEM_SHARED((n_pad, d), jnp.float32),
        ],
    )
    def k(y_hbm, src_hbm, dst_hbm, z_hbm, agg_out,
          src_v, dst_v, buf0, agg_sh):
        c = lax.axis_index("c")
        s = lax.axis_index("s")
        wid = s * _NC + c
        pltpu.sync_copy(src_hbm.at[wid], src_v)
        pltpu.sync_copy(dst_hbm.at[wid], dst_v)
        pltpu.sync_copy(z_hbm.at[pl.ds(s * chunk, chunk)],
                        agg_sh.at[pl.ds(s * chunk, chunk)])
        plsc.subcore_barrier()

        nbc = jnp.where(c == 0, nb_a, nb_b)

        def body(j, carry):
            pltpu.sync_copy(y_hbm.at[src_v.at[j]], buf0)
            pltpu.sync_copy(buf0, agg_sh.at[dst_v.at[j]], add=True)
            return carry

        lax.fori_loop(0, nbc, body, 0)
        plsc.subcore_barrier()
        pltpu.sync_copy(agg_sh.at[pl.ds(s * chunk, chunk)],
                        agg_out.at[c, pl.ds(s * chunk, chunk)])

    return k


def _tc_stage1(X, W1, degp, bn, n_pad):
    n, din = X.shape
    dh = W1.shape[1]
    grid = n_pad // bn

    def body(x_ref, w_ref, deg_ref, xw_ref, y_ref, dis_ref, dinv_ref):
        deg = deg_ref[0] + deg_ref[1] + 1.0
        dis = lax.rsqrt(deg)
        dinv = 1.0 / deg
        xw = jnp.dot(x_ref[...], w_ref[...], preferred_element_type=jnp.float32)
        xw_ref[...] = xw
        y_ref[...] = xw * dis
        dis_ref[...] = dis
        dinv_ref[...] = dinv

    return pl.pallas_call(
        body,
        grid=(grid,),
        in_specs=[
            pl.BlockSpec((bn, din), lambda i: (i, 0)),
            pl.BlockSpec((din, dh), lambda i: (0, 0)),
            pl.BlockSpec((_NC, bn, 1), lambda i: (0, i, 0)),
        ],
        out_specs=[
            pl.BlockSpec((bn, dh), lambda i: (i, 0)),
            pl.BlockSpec((bn, dh), lambda i: (i, 0)),
            pl.BlockSpec((bn, 1), lambda i: (i, 0)),
            pl.BlockSpec((bn, 1), lambda i: (i, 0)),
        ],
        out_shape=[
            jax.ShapeDtypeStruct((n, dh), jnp.float32),
            jax.ShapeDtypeStruct((n_pad, dh), jnp.float32),
            jax.ShapeDtypeStruct((n, 1), jnp.float32),
            jax.ShapeDtypeStruct((n, 1), jnp.float32),
        ],
    )(X, W1, degp)


def _tc_stage2(agg, xw1, dis, dinv, b1, W2, bn, n_pad):
    n, dh = xw1.shape
    do = W2.shape[1]
    grid = n_pad // bn

    def body(a_ref, xw_ref, dis_ref, dinv_ref, b_ref, w_ref, xw2_ref, y2_ref):
        h = (dis_ref[...] * (a_ref[0] + a_ref[1])
             + xw_ref[...] * dinv_ref[...] + b_ref[...])
        h = jnp.maximum(h, 0.0)
        xw2 = jnp.dot(h, w_ref[...], preferred_element_type=jnp.float32)
        xw2_ref[...] = xw2
        y2_ref[...] = xw2 * dis_ref[...]

    return pl.pallas_call(
        body,
        grid=(grid,),
        in_specs=[
            pl.BlockSpec((_NC, bn, dh), lambda i: (0, i, 0)),
            pl.BlockSpec((bn, dh), lambda i: (i, 0)),
            pl.BlockSpec((bn, 1), lambda i: (i, 0)),
            pl.BlockSpec((bn, 1), lambda i: (i, 0)),
            pl.BlockSpec((1, dh), lambda i: (0, 0)),
            pl.BlockSpec((dh, do), lambda i: (0, 0)),
        ],
        out_specs=[
            pl.BlockSpec((bn, do), lambda i: (i, 0)),
            pl.BlockSpec((bn, do), lambda i: (i, 0)),
        ],
        out_shape=[
            jax.ShapeDtypeStruct((n, do), jnp.float32),
            jax.ShapeDtypeStruct((n_pad, do), jnp.float32),
        ],
    )(agg, xw1, dis, dinv, b1, W2)


def _tc_stage3(agg, xw2, dis, dinv, b2, bn, n_pad):
    n, do = xw2.shape
    grid = n_pad // bn

    def body(a_ref, xw_ref, dis_ref, dinv_ref, b_ref, o_ref):
        o_ref[...] = (dis_ref[...] * (a_ref[0] + a_ref[1])
                      + xw_ref[...] * dinv_ref[...] + b_ref[...])

    return pl.pallas_call(
        body,
        grid=(grid,),
        in_specs=[
            pl.BlockSpec((_NC, bn, do), lambda i: (0, i, 0)),
            pl.BlockSpec((bn, do), lambda i: (i, 0)),
            pl.BlockSpec((bn, 1), lambda i: (i, 0)),
            pl.BlockSpec((bn, 1), lambda i: (i, 0)),
            pl.BlockSpec((1, do), lambda i: (0, 0)),
        ],
        out_specs=pl.BlockSpec((bn, do), lambda i: (i, 0)),
        out_shape=jax.ShapeDtypeStruct((n, do), jnp.float32),
    )(agg, xw2, dis, dinv, b2)


def kernel(X, edge_index, W1, b1, W2, b2):
    n, din = X.shape
    e = edge_index.shape[1]
    dh = W1.shape[1]
    do = W2.shape[1]

    nbw = -(-e // (_NW * _B))
    nbw += nbw % 2
    e_padw = _NW * nbw * _B
    per_tile = -(-e // (_NS * _B))
    nb_a = -(-3 * per_tile // 4)
    nb_a += nb_a % 2
    nb_b = max(per_tile - nb_a, 2)
    nb_b += nb_b % 2
    e_a = _NS * nb_a * _B
    e_ab = e_a + _NS * nb_b * _B
    n_pad = -(-(n + 1) // 128) * 128

    src = edge_index[0].astype(jnp.int32)
    dst = edge_index[1].astype(jnp.int32)
    dst_w = jnp.concatenate([dst, jnp.full((e_padw - e,), n, jnp.int32)])
    dst_w = dst_w.reshape(_NW, nbw, _B)
    src_p = jnp.concatenate([src, jnp.zeros((e_ab - e,), jnp.int32)])
    dst_p = jnp.concatenate([dst, jnp.full((e_ab - e,), n, jnp.int32)])

    def widmajor(x, fill):
        xa = x[:e_a].reshape(_NS, 1, nb_a, _B)
        xb = x[e_a:].reshape(_NS, nb_b, _B)
        xb = jnp.concatenate(
            [xb, jnp.full((_NS, nb_a - nb_b, _B), fill, jnp.int32)], axis=1)
        return jnp.concatenate([xa, xb[:, None]], axis=1).reshape(
            _NW, nb_a, _B)

    src_a = widmajor(src_p, 0)
    dst_a = widmajor(dst_p, n)

    zeros_cols = jnp.zeros((n_pad, 16), jnp.float32)
    zeros_rows = jnp.zeros((n_pad, dh), jnp.float32)

    bn = n_pad // _NS

    degp = _sc_degree(dst_w, zeros_cols, n_pad)
    degs = degp[:, :n, 0:1]

    agg_k = _make_sc_aggregate(nb_a, nb_b, dh, n_pad)

    xw1, y1, dis, dinv = _tc_stage1(X, W1, degs, bn, n_pad)
    agg1 = agg_k(y1, src_a, dst_a, zeros_rows)
    xw2, y2 = _tc_stage2(agg1, xw1, dis, dinv, b1.reshape(1, dh), W2, bn, n_pad)
    agg2 = agg_k(y2, src_a, dst_a, zeros_rows)
    return _tc_stage3(agg2, xw2, dis, dinv, b2.reshape(1, do), bn, n_pad)


# speedup vs baseline: 13.3784x; 1.0071x over previous
import functools

import jax
import jax.numpy as jnp
from jax import lax
from jax.experimental import pallas as pl
from jax.experimental.pallas import tpu as pltpu
from jax.experimental.pallas import tpu_sc as plsc

_NC = 2
_NS = 16
_NW = _NC * _NS
_B = 128


def _sc_mesh():
    return plsc.VectorSubcoreMesh(core_axis_name="c", subcore_axis_name="s")


def _sc_degree(dst_idx, zeros_cols, n_pad):
    nb = dst_idx.shape[1]
    chunk = n_pad // _NS

    @functools.partial(
        pl.kernel,
        out_type=jax.ShapeDtypeStruct((_NC, n_pad, 16), jnp.float32),
        mesh=_sc_mesh(),
        scratch_types=[
            pltpu.VMEM((nb, _B), jnp.int32),
            pltpu.VMEM((_B, 16), jnp.float32),
            pltpu.VMEM_SHARED((n_pad, 16), jnp.float32),
        ],
    )
    def k(dst_hbm, z_hbm, deg_out, dst_v, ones_v, deg_sh):
        c = lax.axis_index("c")
        s = lax.axis_index("s")
        wid = s * _NC + c
        pltpu.sync_copy(dst_hbm.at[wid], dst_v)

        def fill(i, carry):
            ones_v[i] = jnp.ones((16,), jnp.float32)
            return carry

        lax.fori_loop(0, _B, fill, 0)
        pltpu.sync_copy(z_hbm.at[pl.ds(s * chunk, chunk)],
                        deg_sh.at[pl.ds(s * chunk, chunk)])
        plsc.subcore_barrier()

        def body(j, carry):
            pltpu.sync_copy(ones_v, deg_sh.at[dst_v.at[j]], add=True)
            return carry

        lax.fori_loop(0, nb, body, 0)
        plsc.subcore_barrier()
        pltpu.sync_copy(deg_sh.at[pl.ds(s * chunk, chunk)],
                        deg_out.at[c, pl.ds(s * chunk, chunk)])

    return k(dst_idx, zeros_cols)


def _make_sc_aggregate(nb_a, nb_b, d, n_pad):
    chunk = n_pad // _NS

    @functools.partial(
        pl.kernel,
        out_type=jax.ShapeDtypeStruct((_NC, n_pad, d), jnp.float32),
        mesh=_sc_mesh(),
        scratch_types=[
            pltpu.VMEM((nb_a, _B), jnp.int32),
            pltpu.VMEM((nb_a, _B), jnp.int32),
            pltpu.VMEM((_B, d), jnp.float32),
            pltpu.VMEM_SHARED((n_pad, d), jnp.float32),
        ],
    )
    def k(y_hbm, src_hbm, dst_hbm, agg_out, src_v, dst_v, buf0, agg_sh):
        c = lax.axis_index("c")
        s = lax.axis_index("s")
        wid = s * _NC + c
        pltpu.sync_copy(src_hbm.at[wid], src_v)
        pltpu.sync_copy(dst_hbm.at[wid], dst_v)

        def zrow(i, carry):
            for kk in range(d // 16):
                buf0[i, pl.ds(kk * 16, 16)] = jnp.zeros((16,), jnp.float32)
            return carry

        lax.fori_loop(0, _B, zrow, 0)
        nfull, rem = chunk // _B, chunk % _B
        for t in range(nfull):
            pltpu.sync_copy(buf0, agg_sh.at[pl.ds(s * chunk + t * _B, _B)])
        if rem:
            pltpu.sync_copy(buf0.at[pl.ds(0, rem)],
                            agg_sh.at[pl.ds(s * chunk + nfull * _B, rem)])
        plsc.subcore_barrier()

        nbc = jnp.where(c == 0, nb_a, nb_b)

        def body(j, carry):
            pltpu.sync_copy(y_hbm.at[src_v.at[j]], buf0)
            pltpu.sync_copy(buf0, agg_sh.at[dst_v.at[j]], add=True)
            return carry

        lax.fori_loop(0, nbc, body, 0)
        plsc.subcore_barrier()
        pltpu.sync_copy(agg_sh.at[pl.ds(s * chunk, chunk)],
                        agg_out.at[c, pl.ds(s * chunk, chunk)])

    return k


def _tc_stage1(X, W1, degp, bn, n_pad):
    n, din = X.shape
    dh = W1.shape[1]
    grid = n_pad // bn

    def body(x_ref, w_ref, deg_ref, xw_ref, y_ref, dis_ref, dinv_ref):
        deg = deg_ref[0] + deg_ref[1] + 1.0
        dis = lax.rsqrt(deg)
        dinv = 1.0 / deg
        xw = jnp.dot(x_ref[...], w_ref[...], preferred_element_type=jnp.float32)
        xw_ref[...] = xw
        y_ref[...] = xw * dis
        dis_ref[...] = dis
        dinv_ref[...] = dinv

    return pl.pallas_call(
        body,
        grid=(grid,),
        in_specs=[
            pl.BlockSpec((bn, din), lambda i: (i, 0)),
            pl.BlockSpec((din, dh), lambda i: (0, 0)),
            pl.BlockSpec((_NC, bn, 1), lambda i: (0, i, 0)),
        ],
        out_specs=[
            pl.BlockSpec((bn, dh), lambda i: (i, 0)),
            pl.BlockSpec((bn, dh), lambda i: (i, 0)),
            pl.BlockSpec((bn, 1), lambda i: (i, 0)),
            pl.BlockSpec((bn, 1), lambda i: (i, 0)),
        ],
        out_shape=[
            jax.ShapeDtypeStruct((n, dh), jnp.float32),
            jax.ShapeDtypeStruct((n_pad, dh), jnp.float32),
            jax.ShapeDtypeStruct((n, 1), jnp.float32),
            jax.ShapeDtypeStruct((n, 1), jnp.float32),
        ],
    )(X, W1, degp)


def _tc_stage2(agg, xw1, dis, dinv, b1, W2, bn, n_pad):
    n, dh = xw1.shape
    do = W2.shape[1]
    grid = n_pad // bn

    def body(a_ref, xw_ref, dis_ref, dinv_ref, b_ref, w_ref, xw2_ref, y2_ref):
        h = (dis_ref[...] * (a_ref[0] + a_ref[1])
             + xw_ref[...] * dinv_ref[...] + b_ref[...])
        h = jnp.maximum(h, 0.0)
        xw2 = jnp.dot(h, w_ref[...], preferred_element_type=jnp.float32)
        xw2_ref[...] = xw2
        y2_ref[...] = xw2 * dis_ref[...]

    return pl.pallas_call(
        body,
        grid=(grid,),
        in_specs=[
            pl.BlockSpec((_NC, bn, dh), lambda i: (0, i, 0)),
            pl.BlockSpec((bn, dh), lambda i: (i, 0)),
            pl.BlockSpec((bn, 1), lambda i: (i, 0)),
            pl.BlockSpec((bn, 1), lambda i: (i, 0)),
            pl.BlockSpec((1, dh), lambda i: (0, 0)),
            pl.BlockSpec((dh, do), lambda i: (0, 0)),
        ],
        out_specs=[
            pl.BlockSpec((bn, do), lambda i: (i, 0)),
            pl.BlockSpec((bn, do), lambda i: (i, 0)),
        ],
        out_shape=[
            jax.ShapeDtypeStruct((n, do), jnp.float32),
            jax.ShapeDtypeStruct((n_pad, do), jnp.float32),
        ],
    )(agg, xw1, dis, dinv, b1, W2)


def _tc_stage3(agg, xw2, dis, dinv, b2, bn, n_pad):
    n, do = xw2.shape
    grid = n_pad // bn

    def body(a_ref, xw_ref, dis_ref, dinv_ref, b_ref, o_ref):
        o_ref[...] = (dis_ref[...] * (a_ref[0] + a_ref[1])
                      + xw_ref[...] * dinv_ref[...] + b_ref[...])

    return pl.pallas_call(
        body,
        grid=(grid,),
        in_specs=[
            pl.BlockSpec((_NC, bn, do), lambda i: (0, i, 0)),
            pl.BlockSpec((bn, do), lambda i: (i, 0)),
            pl.BlockSpec((bn, 1), lambda i: (i, 0)),
            pl.BlockSpec((bn, 1), lambda i: (i, 0)),
            pl.BlockSpec((1, do), lambda i: (0, 0)),
        ],
        out_specs=pl.BlockSpec((bn, do), lambda i: (i, 0)),
        out_shape=jax.ShapeDtypeStruct((n, do), jnp.float32),
    )(agg, xw2, dis, dinv, b2)


def kernel(X, edge_index, W1, b1, W2, b2):
    n, din = X.shape
    e = edge_index.shape[1]
    dh = W1.shape[1]
    do = W2.shape[1]

    nbw = -(-e // (_NW * _B))
    nbw += nbw % 2
    e_padw = _NW * nbw * _B
    per_tile = -(-e // (_NS * _B))
    nb_a = -(-3 * per_tile // 4)
    nb_a += nb_a % 2
    nb_b = max(per_tile - nb_a, 2)
    nb_b += nb_b % 2
    e_a = _NS * nb_a * _B
    e_ab = e_a + _NS * nb_b * _B
    n_pad = -(-(n + 1) // 128) * 128

    src = edge_index[0].astype(jnp.int32)
    dst = edge_index[1].astype(jnp.int32)
    dst_w = jnp.concatenate([dst, jnp.full((e_padw - e,), n, jnp.int32)])
    dst_w = dst_w.reshape(_NW, nbw, _B)
    src_p = jnp.concatenate([src, jnp.zeros((e_ab - e,), jnp.int32)])
    dst_p = jnp.concatenate([dst, jnp.full((e_ab - e,), n, jnp.int32)])

    def widmajor(x, fill):
        xa = x[:e_a].reshape(_NS, 1, nb_a, _B)
        xb = x[e_a:].reshape(_NS, nb_b, _B)
        xb = jnp.concatenate(
            [xb, jnp.full((_NS, nb_a - nb_b, _B), fill, jnp.int32)], axis=1)
        return jnp.concatenate([xa, xb[:, None]], axis=1).reshape(
            _NW, nb_a, _B)

    src_a = widmajor(src_p, 0)
    dst_a = widmajor(dst_p, n)

    zeros_cols = jnp.zeros((n_pad, 16), jnp.float32)

    bn = n_pad // _NS

    degp = _sc_degree(dst_w, zeros_cols, n_pad)
    degs = degp[:, :n, 0:1]

    agg_k = _make_sc_aggregate(nb_a, nb_b, dh, n_pad)

    xw1, y1, dis, dinv = _tc_stage1(X, W1, degs, bn, n_pad)
    agg1 = agg_k(y1, src_a, dst_a)
    xw2, y2 = _tc_stage2(agg1, xw1, dis, dinv, b1.reshape(1, dh), W2, bn, n_pad)
    agg2 = agg_k(y2, src_a, dst_a)
    return _tc_stage3(agg2, xw2, dis, dinv, b2.reshape(1, do), bn, n_pad)
